# Initial kernel scaffold; baseline (speedup 1.0000x reference)
import functools

import jax
import jax.numpy as jnp
from jax.experimental import pallas as pl
from jax.experimental.pallas import tpu as pltpu

_N_CODES = 512
_CODE_DIM = 32
_COMMITMENT = 0.25
_ROWS = 64 * 1024
_BLOCK = 1024
_GRID = _ROWS // _BLOCK


def _vq_body(z_ref, e_ref, zq_ref, idx_ref, loss_ref):
    i = pl.program_id(0)
    z = z_ref[...]
    e = e_ref[...]
    scores = jax.lax.dot_general(
        z, e, (((1,), (1,)), ((), ())),
        preferred_element_type=jnp.float32)
    zsq = jnp.sum(z * z, axis=1, keepdims=True)
    esq = jnp.sum(e * e, axis=1)[None, :]
    dist = zsq + esq - 2.0 * scores
    idx = jnp.argmin(dist, axis=1).astype(jnp.int32)
    idx_ref[0, 0, :] = idx
    onehot = (jax.lax.broadcasted_iota(jnp.int32, (_BLOCK, _N_CODES), 1)
              == idx[:, None]).astype(jnp.float32)
    zq = jax.lax.dot_general(
        onehot, e, (((1,), (0,)), ((), ())),
        preferred_element_type=jnp.float32,
        precision=jax.lax.Precision.HIGHEST)
    zq_ref[...] = z + (zq - z)
    diff = zq - z

    @pl.when(i == 0)
    def _init():
        loss_ref[0, 0] = 0.0

    loss_ref[0, 0] += jnp.sum(diff * diff)


@jax.jit
def _vq(zf, embedding):
    zq, idx, loss = pl.pallas_call(
        _vq_body,
        grid=(_GRID,),
        in_specs=[
            pl.BlockSpec((_BLOCK, _CODE_DIM), lambda i: (i, 0)),
            pl.BlockSpec((_N_CODES, _CODE_DIM), lambda i: (0, 0)),
        ],
        out_specs=[
            pl.BlockSpec((_BLOCK, _CODE_DIM), lambda i: (i, 0)),
            pl.BlockSpec((1, 1, _BLOCK), lambda i: (i, 0, 0)),
            pl.BlockSpec((1, 1), lambda i: (0, 0)),
        ],
        out_shape=[
            jax.ShapeDtypeStruct((_ROWS, _CODE_DIM), jnp.float32),
            jax.ShapeDtypeStruct((_GRID, 1, _BLOCK), jnp.int32),
            jax.ShapeDtypeStruct((1, 1), jnp.float32),
        ],
    )(zf, embedding)
    return zq, idx, loss


def kernel(z, embedding):
    b, n, d = z.shape
    zf = z.reshape(b * n, d)
    zq, idx, loss = _vq(zf, embedding)
    vq_loss = loss[0, 0] * ((1.0 + _COMMITMENT) / (b * n * d))
    return zq.reshape(b, n, d), idx.reshape(b, n), vq_loss


# speedup vs baseline: 1.1359x; 1.1359x over previous
import functools

import jax
import jax.numpy as jnp
from jax.experimental import pallas as pl
from jax.experimental.pallas import tpu as pltpu

_N_CODES = 512
_CODE_DIM = 32
_COMMITMENT = 0.25
_ROWS = 64 * 1024
_BLOCK = 1024
_GRID = _ROWS // _BLOCK


def _vq_body(z_ref, e_ref, zq_ref, idx_ref, loss_ref):
    i = pl.program_id(0)
    z = z_ref[...]
    e = e_ref[...]
    scores = jax.lax.dot_general(
        z.astype(jnp.bfloat16), e.astype(jnp.bfloat16), (((1,), (1,)), ((), ())),
        preferred_element_type=jnp.float32)
    zsq = jnp.sum(z * z, axis=1, keepdims=True)
    esq = jnp.sum(e * e, axis=1)[None, :]
    dist = zsq + esq - 2.0 * scores
    idx = jnp.argmin(dist, axis=1).astype(jnp.int32)
    idx_ref[0, 0, :] = idx
    onehot = (jax.lax.broadcasted_iota(jnp.int32, (_BLOCK, _N_CODES), 1)
              == idx[:, None]).astype(jnp.float32)
    zq = jax.lax.dot_general(
        onehot, e, (((1,), (0,)), ((), ())),
        preferred_element_type=jnp.float32,
        precision=jax.lax.Precision.HIGHEST)
    zq_ref[...] = z + (zq - z)
    diff = zq - z

    @pl.when(i == 0)
    def _init():
        loss_ref[...] = jnp.zeros_like(loss_ref)

    loss_ref[...] += jnp.sum(diff * diff, axis=0, keepdims=True)


@jax.jit
def _vq(zf, embedding):
    zq, idx, loss = pl.pallas_call(
        _vq_body,
        grid=(_GRID,),
        in_specs=[
            pl.BlockSpec((_BLOCK, _CODE_DIM), lambda i: (i, 0)),
            pl.BlockSpec((_N_CODES, _CODE_DIM), lambda i: (0, 0)),
        ],
        out_specs=[
            pl.BlockSpec((_BLOCK, _CODE_DIM), lambda i: (i, 0)),
            pl.BlockSpec((1, 1, _BLOCK), lambda i: (i, 0, 0)),
            pl.BlockSpec((1, _CODE_DIM), lambda i: (0, 0)),
        ],
        out_shape=[
            jax.ShapeDtypeStruct((_ROWS, _CODE_DIM), jnp.float32),
            jax.ShapeDtypeStruct((_GRID, 1, _BLOCK), jnp.int32),
            jax.ShapeDtypeStruct((1, _CODE_DIM), jnp.float32),
        ],
    )(zf, embedding)
    return zq, idx, loss


def kernel(z, embedding):
    b, n, d = z.shape
    zf = z.reshape(b * n, d)
    zq, idx, loss = _vq(zf, embedding)
    vq_loss = jnp.sum(loss) * ((1.0 + _COMMITMENT) / (b * n * d))
    return zq.reshape(b, n, d), idx.reshape(b, n), vq_loss
